# Initial kernel scaffold; baseline (speedup 1.0000x reference)
import jax
import jax.numpy as jnp
from jax.experimental import pallas as pl


def kernel(x, edge_index, batch, root_index, W1, b1, W2, b2):
    raise NotImplementedError("write your pallas kernel here")


# speedup vs baseline: 10.8314x; 10.8314x over previous
import functools

import jax
import jax.numpy as jnp
from jax import lax
from jax.experimental import pallas as pl
from jax.experimental.pallas import tpu as pltpu
from jax.experimental.pallas import tpu_sc as plsc

F32 = jnp.float32
I32 = jnp.int32

NC, NS = 2, 16
CH = 128


def _tbl_rows(n_nodes):
    return ((n_nodes + NS * CH) // (NS * CH)) * (NS * CH)


def _sc_degree(n_nodes, e_pad):
    per_tile = e_pad // (NC * NS)
    nchunk = per_tile // CH
    rows = _tbl_rows(n_nodes) // 128

    @functools.partial(
        pl.kernel,
        out_type=jax.ShapeDtypeStruct((NC, rows, 128), F32),
        mesh=plsc.VectorSubcoreMesh(core_axis_name="c", subcore_axis_name="s"),
        compiler_params=pltpu.CompilerParams(needs_layout_passes=False),
        scratch_types=[
            pltpu.VMEM((CH,), I32),
            pltpu.VMEM((rows, 128), F32),
            pltpu.VMEM((rows,), I32),
            pltpu.VMEM_SHARED((rows, 128), F32),
            pltpu.SemaphoreType.DMA,
        ],
    )
    def deg_kernel(dst_hbm, out_hbm, idx_v, deg_v, iota_v, deg_sh, sem):
        c = lax.axis_index("c")
        s = lax.axis_index("s")

        def zf(r, _):
            for j in range(8):
                deg_v[r, pl.ds(j * 16, 16)] = jnp.zeros((16,), F32)
            return ()
        lax.fori_loop(0, rows, zf, ())

        def iof(k, _):
            iota_v[pl.ds(k * 16, 16)] = lax.iota(I32, 16) + k * 16
            return ()
        lax.fori_loop(0, rows // 16, iof, ())

        @pl.when(s == 0)
        def _():
            pltpu.sync_copy(deg_v, deg_sh)
        plsc.subcore_barrier()

        base = c * (e_pad // NC) + s * per_tile

        def body(k, _):
            pltpu.sync_copy(dst_hbm.at[pl.ds(base + k * CH, CH)], idx_v)
            for j in range(CH // 16):
                i16 = idx_v[pl.ds(j * 16, 16)]
                plsc.addupdate_scatter(
                    deg_v,
                    [lax.shift_right_logical(i16, 7),
                     lax.bitwise_and(i16, 127)],
                    jnp.ones((16,), F32))
            return ()
        lax.fori_loop(0, nchunk, body, ())
        pltpu.async_copy(deg_v, deg_sh.at[iota_v], sem, add=True).wait()
        plsc.subcore_barrier()

        @pl.when(s < 10)
        def _():
            pltpu.sync_copy(deg_sh.at[pl.ds(s * 8, 8)],
                            out_hbm.at[c, pl.ds(s * 8, 8)])

    return deg_kernel


def _sc_propagate(n_nodes, e_pad, d):
    per_tile = e_pad // (NC * NS)
    nchunk = per_tile // CH
    tbl_rows = _tbl_rows(n_nodes)
    per_tile_out = tbl_rows // NS

    @functools.partial(
        pl.kernel,
        out_type=jax.ShapeDtypeStruct((NC, tbl_rows, d), F32),
        mesh=plsc.VectorSubcoreMesh(core_axis_name="c", subcore_axis_name="s"),
        scratch_types=[
            pltpu.VMEM((CH,), I32),
            pltpu.VMEM((CH,), I32),
            pltpu.VMEM((CH, d), F32),
            pltpu.VMEM((CH, d), F32),
            pltpu.VMEM_SHARED((tbl_rows, d), F32),
            pltpu.SemaphoreType.DMA,
            pltpu.SemaphoreType.DMA,
        ],
    )
    def prop_kernel(g_hbm, src_hbm, dst_hbm, acc_hbm,
                    src_v, dst_v, rows_v, zeros_v, acc_sh, gsem, ssem):
        c = lax.axis_index("c")
        s = lax.axis_index("s")

        def zfill(r, _):
            for j in range(d // 16):
                zeros_v[r, pl.ds(j * 16, 16)] = jnp.zeros((16,), F32)
            return ()
        lax.fori_loop(0, CH, zfill, ())

        zcopies = tbl_rows // (NS * CH)

        def zcopy(k, _):
            pltpu.sync_copy(
                zeros_v, acc_sh.at[pl.ds((s * zcopies + k) * CH, CH)])
            return ()
        lax.fori_loop(0, zcopies, zcopy, ())
        plsc.subcore_barrier()

        base = c * (e_pad // NC) + s * per_tile

        def body(k, _):
            pltpu.sync_copy(src_hbm.at[pl.ds(base + k * CH, CH)], src_v)
            pltpu.sync_copy(dst_hbm.at[pl.ds(base + k * CH, CH)], dst_v)
            pltpu.async_copy(g_hbm.at[src_v], rows_v, gsem).wait()
            pltpu.async_copy(rows_v, acc_sh.at[dst_v], ssem, add=True).wait()
            return ()
        lax.fori_loop(0, nchunk, body, ())
        plsc.subcore_barrier()

        pltpu.sync_copy(
            acc_sh.at[pl.ds(s * per_tile_out, per_tile_out)],
            acc_hbm.at[c, pl.ds(s * per_tile_out, per_tile_out)],
        )

    return prop_kernel


def _dinv(d0_ref, d1_ref):
    return lax.rsqrt(d0_ref[...] + d1_ref[...] + 1.0)


def _tc_g1(n, d, blk):
    def body(x_ref, w_ref, d0_ref, d1_ref, g_ref):
        h = jnp.dot(x_ref[...], w_ref[...], preferred_element_type=F32)
        g_ref[...] = h * _dinv(d0_ref, d1_ref)

    grid = (n // blk,)
    return pl.pallas_call(
        body,
        grid=grid,
        in_specs=[
            pl.BlockSpec((blk, d), lambda i: (i, 0)),
            pl.BlockSpec((d, d), lambda i: (0, 0)),
            pl.BlockSpec((blk, 1), lambda i: (i, 0)),
            pl.BlockSpec((blk, 1), lambda i: (i, 0)),
        ],
        out_specs=pl.BlockSpec((blk, d), lambda i: (i, 0)),
        out_shape=jax.ShapeDtypeStruct((n, d), F32),
    )


def _tc_root_table(n, d, g, with_matmul):
    if with_matmul:
        def body(_, x_ref, w_ref, out_ref):
            r = jnp.maximum(x_ref[0], 0.0)
            out_ref[0] = jnp.dot(r, w_ref[...], preferred_element_type=F32)
        in_specs = [
            pl.BlockSpec((1, 1, d), lambda i, root: (root[i], 0, 0)),
            pl.BlockSpec((d, d), lambda i, root: (0, 0)),
        ]
    else:
        def body(_, x_ref, out_ref):
            out_ref[...] = x_ref[...]
        in_specs = [pl.BlockSpec((1, 1, d), lambda i, root: (root[i], 0, 0))]

    call = pl.pallas_call(
        body,
        grid_spec=pltpu.PrefetchScalarGridSpec(
            num_scalar_prefetch=1,
            grid=(g,),
            in_specs=in_specs,
            out_specs=pl.BlockSpec((1, 1, d), lambda i, root: (i, 0, 0)),
        ),
        out_shape=jax.ShapeDtypeStruct((g, 1, d), F32),
    )

    if with_matmul:
        def run(root, x, w):
            return call(root, x.reshape(n, 1, d), w).reshape(g, d)
    else:
        def run(root, x):
            return call(root, x.reshape(n, 1, d)).reshape(g, d)
    return run


def _tc_layer2_in(n, d, g, blk):
    def body(acc_ref, g1_ref, d0_ref, d1_ref, batch_ref, rtab_ref, w2a_ref,
             b1_ref, out1_ref, g2_ref):
        dinv = _dinv(d0_ref, d1_ref)
        out1 = dinv * (acc_ref[0] + acc_ref[1] + g1_ref[...]) + b1_ref[...]
        out1_ref[...] = out1
        t = jnp.dot(jnp.maximum(out1, 0.0), w2a_ref[...],
                    preferred_element_type=F32)
        gids = lax.broadcasted_iota(I32, (1, g), 1)
        oh = (batch_ref[...] == gids).astype(F32)
        t = t + jnp.dot(oh, rtab_ref[...], preferred_element_type=F32)
        g2_ref[...] = dinv * t

    grid = (n // blk,)
    return pl.pallas_call(
        body,
        grid=grid,
        in_specs=[
            pl.BlockSpec((NC, blk, d), lambda i: (0, i, 0)),
            pl.BlockSpec((blk, d), lambda i: (i, 0)),
            pl.BlockSpec((blk, 1), lambda i: (i, 0)),
            pl.BlockSpec((blk, 1), lambda i: (i, 0)),
            pl.BlockSpec((blk, 1), lambda i: (i, 0)),
            pl.BlockSpec((g, d), lambda i: (0, 0)),
            pl.BlockSpec((d, d), lambda i: (0, 0)),
            pl.BlockSpec((1, d), lambda i: (0, 0)),
        ],
        out_specs=[
            pl.BlockSpec((blk, d), lambda i: (i, 0)),
            pl.BlockSpec((blk, d), lambda i: (i, 0)),
        ],
        out_shape=[
            jax.ShapeDtypeStruct((n, d), F32),
            jax.ShapeDtypeStruct((n, d), F32),
        ],
    )


def _tc_final(n, d, g, blk):
    nblk = n // blk

    def body(acc_ref, g2_ref, d0_ref, d1_ref, batch_ref, rtab2_ref, b2_ref,
             left_ref, right_ref, sums_ref, counts_ref):
        i = pl.program_id(0)

        @pl.when(i == 0)
        def _():
            sums_ref[...] = jnp.zeros_like(sums_ref)
            counts_ref[...] = jnp.zeros_like(counts_ref)

        dinv = _dinv(d0_ref, d1_ref)
        h3 = jnp.maximum(
            dinv * (acc_ref[0] + acc_ref[1] + g2_ref[...]) + b2_ref[...], 0.0)
        gids = lax.broadcasted_iota(I32, (1, g), 1)
        oh = (batch_ref[...] == gids).astype(F32)
        dn = (((0,), (0,)), ((), ()))
        sums_ref[...] += lax.dot_general(
            oh, h3, dn, preferred_element_type=F32)
        counts_ref[...] += lax.dot_general(
            oh, jnp.ones((blk, d), F32), dn, preferred_element_type=F32)

        @pl.when(i == nblk - 1)
        def _():
            cnt = counts_ref[...]
            left_ref[...] = sums_ref[...] / jnp.maximum(cnt, 1.0)
            right_ref[...] = jnp.where(cnt > 0.0, rtab2_ref[...], 0.0)

    return pl.pallas_call(
        body,
        grid=(nblk,),
        in_specs=[
            pl.BlockSpec((NC, blk, d), lambda i: (0, i, 0)),
            pl.BlockSpec((blk, d), lambda i: (i, 0)),
            pl.BlockSpec((blk, 1), lambda i: (i, 0)),
            pl.BlockSpec((blk, 1), lambda i: (i, 0)),
            pl.BlockSpec((blk, 1), lambda i: (i, 0)),
            pl.BlockSpec((g, d), lambda i: (0, 0)),
            pl.BlockSpec((1, d), lambda i: (0, 0)),
        ],
        out_specs=[
            pl.BlockSpec((g, d), lambda i: (0, 0)),
            pl.BlockSpec((g, d), lambda i: (0, 0)),
        ],
        out_shape=[
            jax.ShapeDtypeStruct((g, d), F32),
            jax.ShapeDtypeStruct((g, d), F32),
        ],
        scratch_shapes=[
            pltpu.VMEM((g, d), F32),
            pltpu.VMEM((g, d), F32),
        ],
    )


@jax.jit
def kernel(x, edge_index, batch, root_index, W1, b1, W2, b2):
    n, d = x.shape
    e = edge_index.shape[1]
    g = root_index.shape[0]
    blk = 1000

    chunk_all = NC * NS * CH
    e_pad = ((e + chunk_all - 1) // chunk_all) * chunk_all

    src = edge_index[0].astype(I32)
    dst = edge_index[1].astype(I32)
    pad = e_pad - e
    src_p = jnp.concatenate([src, jnp.zeros((pad,), I32)])
    dst_p = jnp.concatenate([dst, jnp.full((pad,), n, I32)])
    batch2d = batch.astype(I32).reshape(n, 1)
    root_i = root_index.astype(I32)
    b1r = b1.reshape(1, d).astype(F32)
    b2r = b2.reshape(1, d).astype(F32)
    w2a = W2[:d].astype(F32)
    w2b = W2[d:].astype(F32)
    xf = x.astype(F32)

    degs = _sc_degree(n, e_pad)(dst_p)
    deg0 = degs[0].reshape(-1)[:n].reshape(n, 1)
    deg1 = degs[1].reshape(-1)[:n].reshape(n, 1)

    g1 = _tc_g1(n, d, blk)(xf, W1.astype(F32), deg0, deg1)
    acc1 = _sc_propagate(n, e_pad, d)(g1, src_p, dst_p)

    rtab = _tc_root_table(n, d, g, True)(root_i, xf, w2b)
    out1, g2 = _tc_layer2_in(n, d, g, blk)(
        acc1, g1, deg0, deg1, batch2d, rtab, w2a, b1r)

    acc2 = _sc_propagate(n, e_pad, d)(g2, src_p, dst_p)
    rtab2 = _tc_root_table(n, d, g, False)(root_i, out1)

    left, right = _tc_final(n, d, g, blk)(
        acc2, g2, deg0, deg1, batch2d, rtab2, b2r)
    return jnp.concatenate([left, right], axis=1)


# speedup vs baseline: 11.5465x; 1.0660x over previous
import functools

import jax
import jax.numpy as jnp
from jax import lax
from jax.experimental import pallas as pl
from jax.experimental.pallas import tpu as pltpu
from jax.experimental.pallas import tpu_sc as plsc

F32 = jnp.float32
I32 = jnp.int32

NC, NS = 2, 16
CH = 128


def _tbl_rows(n_nodes):
    return ((n_nodes + NS * CH) // (NS * CH)) * (NS * CH)


def _sc_degree(n_nodes, e_pad):
    per_tile = e_pad // (NC * NS)
    nchunk = per_tile // CH
    rows = _tbl_rows(n_nodes) // 128

    @functools.partial(
        pl.kernel,
        out_type=jax.ShapeDtypeStruct((NC, rows, 128), F32),
        mesh=plsc.VectorSubcoreMesh(core_axis_name="c", subcore_axis_name="s"),
        compiler_params=pltpu.CompilerParams(needs_layout_passes=False),
        scratch_types=[
            pltpu.VMEM((CH,), I32),
            pltpu.VMEM((rows, 128), F32),
            pltpu.VMEM((rows,), I32),
            pltpu.VMEM_SHARED((rows, 128), F32),
            pltpu.SemaphoreType.DMA,
        ],
    )
    def deg_kernel(dst_hbm, out_hbm, idx_v, deg_v, iota_v, deg_sh, sem):
        c = lax.axis_index("c")
        s = lax.axis_index("s")

        def zf(r, _):
            for j in range(8):
                deg_v[r, pl.ds(j * 16, 16)] = jnp.zeros((16,), F32)
            return ()
        lax.fori_loop(0, rows, zf, ())

        def iof(k, _):
            iota_v[pl.ds(k * 16, 16)] = lax.iota(I32, 16) + k * 16
            return ()
        lax.fori_loop(0, rows // 16, iof, ())

        @pl.when(s == 0)
        def _():
            pltpu.sync_copy(deg_v, deg_sh)
        plsc.subcore_barrier()

        base = c * (e_pad // NC) + s * per_tile

        def body(k, _):
            pltpu.sync_copy(dst_hbm.at[pl.ds(base + k * CH, CH)], idx_v)
            for j in range(CH // 16):
                i16 = idx_v[pl.ds(j * 16, 16)]
                plsc.addupdate_scatter(
                    deg_v,
                    [lax.shift_right_logical(i16, 7),
                     lax.bitwise_and(i16, 127)],
                    jnp.ones((16,), F32))
            return ()
        lax.fori_loop(0, nchunk, body, ())
        pltpu.async_copy(deg_v, deg_sh.at[iota_v], sem, add=True).wait()
        plsc.subcore_barrier()

        @pl.when(s < 10)
        def _():
            pltpu.sync_copy(deg_sh.at[pl.ds(s * 8, 8)],
                            out_hbm.at[c, pl.ds(s * 8, 8)])

    return deg_kernel


def _sc_propagate(n_nodes, e_pad, d):
    per_tile = e_pad // (NC * NS)
    nchunk = per_tile // CH
    assert nchunk % 4 == 0 and nchunk >= 8
    tbl_rows = _tbl_rows(n_nodes)
    per_tile_out = tbl_rows // NS

    @functools.partial(
        pl.kernel,
        out_type=jax.ShapeDtypeStruct((NC, tbl_rows, d), F32),
        mesh=plsc.VectorSubcoreMesh(core_axis_name="c", subcore_axis_name="s"),
        scratch_types=[
            [pltpu.VMEM((CH,), I32)] * 2,
            [pltpu.VMEM((CH,), I32)] * 4,
            [pltpu.VMEM((CH, d), F32)] * 2,
            pltpu.VMEM_SHARED((tbl_rows, d), F32),
            [pltpu.SemaphoreType.DMA] * 2,
            [pltpu.SemaphoreType.DMA] * 2,
            [pltpu.SemaphoreType.DMA] * 2,
            [pltpu.SemaphoreType.DMA] * 4,
        ],
    )
    def prop_kernel(g_hbm, src_hbm, dst_hbm, acc_hbm,
                    srcb, dstb, rows_v, acc_sh, gsem, ssem, srcsem, dsem):
        c = lax.axis_index("c")
        s = lax.axis_index("s")
        base = c * (e_pad // NC) + s * per_tile

        def eoff(k):
            return pl.multiple_of(base + k * CH, CH)

        def load_src(k, sl):
            pltpu.async_copy(
                src_hbm.at[pl.ds(eoff(k), CH)], srcb[sl], srcsem[sl])

        def wait_src(k, sl):
            pltpu.make_async_copy(
                src_hbm.at[pl.ds(eoff(k), CH)], srcb[sl], srcsem[sl]).wait()

        def load_dst(k, sl):
            pltpu.async_copy(
                dst_hbm.at[pl.ds(eoff(k), CH)], dstb[sl], dsem[sl])

        def wait_dst(k, sl):
            pltpu.make_async_copy(
                dst_hbm.at[pl.ds(eoff(k), CH)], dstb[sl], dsem[sl]).wait()

        def start_gather(b):
            pltpu.async_copy(g_hbm.at[srcb[b]], rows_v[b], gsem[b])

        def wait_gather(b):
            pltpu.make_async_copy(g_hbm.at[srcb[b]], rows_v[b], gsem[b]).wait()

        def start_scatter(b, b4):
            pltpu.async_copy(
                rows_v[b], acc_sh.at[dstb[b4]], ssem[b], add=True)

        def wait_scatter(b, b4):
            pltpu.make_async_copy(
                rows_v[b], acc_sh.at[dstb[b4]], ssem[b]).wait()

        def zfill(r, _):
            for j in range(d // 16):
                rows_v[0][r, pl.ds(j * 16, 16)] = jnp.zeros((16,), F32)
            return ()
        lax.fori_loop(0, CH, zfill, ())
        zcopies = tbl_rows // (NS * CH)

        def zcopy(k, _):
            pltpu.sync_copy(
                rows_v[0], acc_sh.at[pl.ds((s * zcopies + k) * CH, CH)])
            return ()
        lax.fori_loop(0, zcopies, zcopy, ())
        plsc.subcore_barrier()

        pltpu.sync_copy(src_hbm.at[pl.ds(eoff(0), CH)], srcb[0])
        pltpu.sync_copy(src_hbm.at[pl.ds(eoff(1), CH)], srcb[1])
        for k in range(4):
            pltpu.sync_copy(dst_hbm.at[pl.ds(eoff(k), CH)], dstb[k])
        start_gather(0)
        start_gather(1)
        wait_gather(0)
        start_scatter(0, 0)
        load_src(2, 0)
        wait_scatter(0, 0)
        load_dst(4, 0)
        wait_src(2, 0)
        start_gather(0)
        wait_gather(1)
        start_scatter(1, 1)
        load_src(3, 1)
        wait_scatter(1, 1)
        load_dst(5, 1)
        wait_src(3, 1)
        start_gather(1)
        wait_gather(0)
        start_scatter(0, 2)
        load_src(4, 0)

        def body(kg, _):
            for j in range(4):
                k = kg * 4 + j
                b = j % 2
                pb = 1 - b
                b4 = j
                wait_scatter(b, (j + 2) % 4)
                load_dst(k + 2, (j + 2) % 4)
                wait_src(k, b)
                wait_dst(k, b4)
                start_gather(b)
                wait_gather(pb)
                start_scatter(pb, (j + 3) % 4)
                load_src(k + 1, pb)
            return ()
        lax.fori_loop(1, nchunk // 4, body, ())

        wait_gather(1)
        start_scatter(1, 3)
        wait_scatter(0, 2)
        wait_scatter(1, 3)
        wait_src(nchunk, nchunk % 2)
        wait_dst(nchunk, nchunk % 4)
        wait_dst(nchunk + 1, (nchunk + 1) % 4)
        plsc.subcore_barrier()

        pltpu.sync_copy(
            acc_sh.at[pl.ds(s * per_tile_out, per_tile_out)],
            acc_hbm.at[c, pl.ds(s * per_tile_out, per_tile_out)],
        )

    return prop_kernel


def _dinv(d0_ref, d1_ref):
    return lax.rsqrt(d0_ref[...] + d1_ref[...] + 1.0)


def _tc_g1(n, d, blk):
    def body(x_ref, w_ref, d0_ref, d1_ref, g_ref):
        h = jnp.dot(x_ref[...], w_ref[...], preferred_element_type=F32)
        g_ref[...] = h * _dinv(d0_ref, d1_ref)

    grid = (n // blk,)
    return pl.pallas_call(
        body,
        grid=grid,
        in_specs=[
            pl.BlockSpec((blk, d), lambda i: (i, 0)),
            pl.BlockSpec((d, d), lambda i: (0, 0)),
            pl.BlockSpec((blk, 1), lambda i: (i, 0)),
            pl.BlockSpec((blk, 1), lambda i: (i, 0)),
        ],
        out_specs=pl.BlockSpec((blk, d), lambda i: (i, 0)),
        out_shape=jax.ShapeDtypeStruct((n, d), F32),
    )


def _tc_root_table(n, d, g, with_matmul):
    if with_matmul:
        def body(_, x_ref, w_ref, out_ref):
            r = jnp.maximum(x_ref[0], 0.0)
            out_ref[0] = jnp.dot(r, w_ref[...], preferred_element_type=F32)
        in_specs = [
            pl.BlockSpec((1, 1, d), lambda i, root: (root[i], 0, 0)),
            pl.BlockSpec((d, d), lambda i, root: (0, 0)),
        ]
    else:
        def body(_, x_ref, out_ref):
            out_ref[...] = x_ref[...]
        in_specs = [pl.BlockSpec((1, 1, d), lambda i, root: (root[i], 0, 0))]

    call = pl.pallas_call(
        body,
        grid_spec=pltpu.PrefetchScalarGridSpec(
            num_scalar_prefetch=1,
            grid=(g,),
            in_specs=in_specs,
            out_specs=pl.BlockSpec((1, 1, d), lambda i, root: (i, 0, 0)),
        ),
        out_shape=jax.ShapeDtypeStruct((g, 1, d), F32),
    )

    if with_matmul:
        def run(root, x, w):
            return call(root, x.reshape(n, 1, d), w).reshape(g, d)
    else:
        def run(root, x):
            return call(root, x.reshape(n, 1, d)).reshape(g, d)
    return run


def _tc_layer2_in(n, d, g, blk):
    def body(acc_ref, g1_ref, d0_ref, d1_ref, batch_ref, rtab_ref, w2a_ref,
             b1_ref, out1_ref, g2_ref):
        dinv = _dinv(d0_ref, d1_ref)
        out1 = dinv * (acc_ref[0] + acc_ref[1] + g1_ref[...]) + b1_ref[...]
        out1_ref[...] = out1
        t = jnp.dot(jnp.maximum(out1, 0.0), w2a_ref[...],
                    preferred_element_type=F32)
        gids = lax.broadcasted_iota(I32, (1, g), 1)
        oh = (batch_ref[...] == gids).astype(F32)
        t = t + jnp.dot(oh, rtab_ref[...], preferred_element_type=F32)
        g2_ref[...] = dinv * t

    grid = (n // blk,)
    return pl.pallas_call(
        body,
        grid=grid,
        in_specs=[
            pl.BlockSpec((NC, blk, d), lambda i: (0, i, 0)),
            pl.BlockSpec((blk, d), lambda i: (i, 0)),
            pl.BlockSpec((blk, 1), lambda i: (i, 0)),
            pl.BlockSpec((blk, 1), lambda i: (i, 0)),
            pl.BlockSpec((blk, 1), lambda i: (i, 0)),
            pl.BlockSpec((g, d), lambda i: (0, 0)),
            pl.BlockSpec((d, d), lambda i: (0, 0)),
            pl.BlockSpec((1, d), lambda i: (0, 0)),
        ],
        out_specs=[
            pl.BlockSpec((blk, d), lambda i: (i, 0)),
            pl.BlockSpec((blk, d), lambda i: (i, 0)),
        ],
        out_shape=[
            jax.ShapeDtypeStruct((n, d), F32),
            jax.ShapeDtypeStruct((n, d), F32),
        ],
    )


def _tc_final(n, d, g, blk):
    nblk = n // blk

    def body(acc_ref, g2_ref, d0_ref, d1_ref, batch_ref, rtab2_ref, b2_ref,
             left_ref, right_ref, sums_ref, counts_ref):
        i = pl.program_id(0)

        @pl.when(i == 0)
        def _():
            sums_ref[...] = jnp.zeros_like(sums_ref)
            counts_ref[...] = jnp.zeros_like(counts_ref)

        dinv = _dinv(d0_ref, d1_ref)
        h3 = jnp.maximum(
            dinv * (acc_ref[0] + acc_ref[1] + g2_ref[...]) + b2_ref[...], 0.0)
        gids = lax.broadcasted_iota(I32, (1, g), 1)
        oh = (batch_ref[...] == gids).astype(F32)
        dn = (((0,), (0,)), ((), ()))
        sums_ref[...] += lax.dot_general(
            oh, h3, dn, preferred_element_type=F32)
        counts_ref[...] += lax.dot_general(
            oh, jnp.ones((blk, d), F32), dn, preferred_element_type=F32)

        @pl.when(i == nblk - 1)
        def _():
            cnt = counts_ref[...]
            left_ref[...] = sums_ref[...] / jnp.maximum(cnt, 1.0)
            right_ref[...] = jnp.where(cnt > 0.0, rtab2_ref[...], 0.0)

    return pl.pallas_call(
        body,
        grid=(nblk,),
        in_specs=[
            pl.BlockSpec((NC, blk, d), lambda i: (0, i, 0)),
            pl.BlockSpec((blk, d), lambda i: (i, 0)),
            pl.BlockSpec((blk, 1), lambda i: (i, 0)),
            pl.BlockSpec((blk, 1), lambda i: (i, 0)),
            pl.BlockSpec((blk, 1), lambda i: (i, 0)),
            pl.BlockSpec((g, d), lambda i: (0, 0)),
            pl.BlockSpec((1, d), lambda i: (0, 0)),
        ],
        out_specs=[
            pl.BlockSpec((g, d), lambda i: (0, 0)),
            pl.BlockSpec((g, d), lambda i: (0, 0)),
        ],
        out_shape=[
            jax.ShapeDtypeStruct((g, d), F32),
            jax.ShapeDtypeStruct((g, d), F32),
        ],
        scratch_shapes=[
            pltpu.VMEM((g, d), F32),
            pltpu.VMEM((g, d), F32),
        ],
    )


@jax.jit
def kernel(x, edge_index, batch, root_index, W1, b1, W2, b2):
    n, d = x.shape
    e = edge_index.shape[1]
    g = root_index.shape[0]
    blk = 1000

    chunk_all = NC * NS * CH * 4
    e_pad = ((e + chunk_all - 1) // chunk_all) * chunk_all

    src = edge_index[0].astype(I32)
    dst = edge_index[1].astype(I32)
    pad = e_pad - e
    src_p = jnp.concatenate([src, jnp.zeros((pad + 512,), I32)])
    dst_p = jnp.concatenate(
        [dst, jnp.full((pad + 512,), n, I32)])
    batch2d = batch.astype(I32).reshape(n, 1)
    root_i = root_index.astype(I32)
    b1r = b1.reshape(1, d).astype(F32)
    b2r = b2.reshape(1, d).astype(F32)
    w2a = W2[:d].astype(F32)
    w2b = W2[d:].astype(F32)
    xf = x.astype(F32)

    degs = _sc_degree(n, e_pad)(dst_p)
    deg0 = degs[0].reshape(-1)[:n].reshape(n, 1)
    deg1 = degs[1].reshape(-1)[:n].reshape(n, 1)

    g1 = _tc_g1(n, d, blk)(xf, W1.astype(F32), deg0, deg1)
    acc1 = _sc_propagate(n, e_pad, d)(g1, src_p, dst_p)

    rtab = _tc_root_table(n, d, g, True)(root_i, xf, w2b)
    out1, g2 = _tc_layer2_in(n, d, g, blk)(
        acc1, g1, deg0, deg1, batch2d, rtab, w2a, b1r)

    acc2 = _sc_propagate(n, e_pad, d)(g2, src_p, dst_p)
    rtab2 = _tc_root_table(n, d, g, False)(root_i, out1)

    left, right = _tc_final(n, d, g, blk)(
        acc2, g2, deg0, deg1, batch2d, rtab2, b2r)
    return jnp.concatenate([left, right], axis=1)
